# baseline (speedup 1.0000x reference)
import functools

import jax
import jax.numpy as jnp
from jax import lax
from jax.experimental import pallas as pl
from jax.experimental.pallas import tpu as pltpu
from jax.experimental.pallas import tpu_sc as plsc

N = 10000
E = 320000
C = 128
OUT = 128

_NC = 2
_NS = 16
_NW = _NC * _NS
CHUNK = 128
_BLK = 8
_NB = 10
_CH = _NW * _NB * _BLK
_EP = _CH * CHUNK
_PAD = _EP - E
_CHU = E // CHUNK
_CHBASE = _CHU // _NW
_CHR = _CHU % _NW
_RT = N // _NS
_RZ = (N + 16) // _NS


def _counts_body(e0_hbm, e1_hbm, z_hbm, ones_hbm,
                 cs0_hbm, cs1_hbm, cd0_hbm, cd1_hbm,
                 i0_v, i1_v, ones_v, cs_sh, cd_sh):
    cid = lax.axis_index("c")
    sid = lax.axis_index("s")
    wid = sid * _NC + cid
    r0 = sid * _RT
    pltpu.sync_copy(ones_hbm, ones_v)
    pltpu.sync_copy(z_hbm, cs_sh.at[pl.ds(r0, _RT)])
    pltpu.sync_copy(z_hbm, cd_sh.at[pl.ds(r0, _RT)])
    plsc.subcore_barrier()
    nch = jnp.where(wid < _CHR, _CHBASE + 1, _CHBASE)

    def _chunk(j, carry):
        off = (j * _NW + wid) * CHUNK
        pltpu.sync_copy(e0_hbm.at[pl.ds(off, CHUNK)], i0_v)
        pltpu.sync_copy(e1_hbm.at[pl.ds(off, CHUNK)], i1_v)
        pltpu.sync_copy(ones_v, cs_sh.at[i0_v], add=True)
        pltpu.sync_copy(ones_v, cd_sh.at[i1_v], add=True)
        return carry

    lax.fori_loop(0, nch, _chunk, 0)
    plsc.subcore_barrier()
    ro = sid * _RT

    @pl.when(cid == 0)
    def _():
        pltpu.sync_copy(cs_sh.at[pl.ds(ro, _RT)], cs0_hbm.at[pl.ds(ro, _RT)])
        pltpu.sync_copy(cd_sh.at[pl.ds(ro, _RT)], cd0_hbm.at[pl.ds(ro, _RT)])

    @pl.when(cid == 1)
    def _():
        pltpu.sync_copy(cs_sh.at[pl.ds(ro, _RT)], cs1_hbm.at[pl.ds(ro, _RT)])
        pltpu.sync_copy(cd_sh.at[pl.ds(ro, _RT)], cd1_hbm.at[pl.ds(ro, _RT)])


_counts_kernel = functools.partial(
    pl.kernel,
    out_type=[jax.ShapeDtypeStruct((N, 16), jnp.float32)] * 4,
    mesh=plsc.VectorSubcoreMesh(core_axis_name="c", subcore_axis_name="s"),
    compiler_params=pltpu.CompilerParams(use_tc_tiling_on_sc=False,
                                         needs_layout_passes=False),
    scratch_types=[
        pltpu.VMEM((CHUNK,), jnp.int32),
        pltpu.VMEM((CHUNK,), jnp.int32),
        pltpu.VMEM((CHUNK, 16), jnp.float32),
        pltpu.VMEM_SHARED((N, 16), jnp.float32),
        pltpu.VMEM_SHARED((N, 16), jnp.float32),
    ],
)(_counts_body)


def _agg_body(e0_hbm, e1_hbm, t_hbm, x_hbm, z_hbm,
              a0_hbm, a1_hbm,
              e0_v, tgt_v, q0, q1, r0_v, r1_v,
              t_v, agg_sh, s0, s1):
    qs = (q0, q1)
    rs = (r0_v, r1_v)
    sems = (s0, s1)
    cid = lax.axis_index("c")
    sid = lax.axis_index("s")
    wid = sid * _NC + cid
    pltpu.sync_copy(t_hbm, t_v)
    pltpu.sync_copy(z_hbm, agg_sh.at[pl.ds(sid * _RZ, _RZ)])
    plsc.subcore_barrier()

    def _block(ib, carry):
        def _off(j):
            return ((ib * _BLK + j) * _NW + wid) * CHUNK

        cps = [None] * _BLK
        pltpu.sync_copy(e1_hbm.at[pl.ds(_off(0), CHUNK)], qs[0])
        cps[0] = pltpu.async_copy(x_hbm.at[qs[0]], rs[0], sems[0])
        for j in range(_BLK):
            if j + 1 < _BLK:
                k = (j + 1) % 2
                pltpu.sync_copy(e1_hbm.at[pl.ds(_off(j + 1), CHUNK)], qs[k])
                cps[j + 1] = pltpu.async_copy(x_hbm.at[qs[k]], rs[k], sems[k])
            pltpu.sync_copy(e0_hbm.at[pl.ds(_off(j), CHUNK)], e0_v)

            def _map(i, c2):
                idx16 = e0_v[pl.ds(i * 16, 16)]
                tgt_v[pl.ds(i * 16, 16)] = plsc.load_gather(t_v, [idx16])
                return c2

            lax.fori_loop(0, CHUNK // 16, _map, 0)
            cps[j].wait()
            pltpu.sync_copy(rs[j % 2], agg_sh.at[tgt_v], add=True)
        return carry

    lax.fori_loop(0, _NB, _block, 0)
    plsc.subcore_barrier()
    ro = sid * _RT

    @pl.when(cid == 0)
    def _():
        pltpu.sync_copy(agg_sh.at[pl.ds(ro, _RT)], a0_hbm.at[pl.ds(ro, _RT)])

    @pl.when(cid == 1)
    def _():
        pltpu.sync_copy(agg_sh.at[pl.ds(ro, _RT)], a1_hbm.at[pl.ds(ro, _RT)])


_agg_kernel = functools.partial(
    pl.kernel,
    out_type=[jax.ShapeDtypeStruct((N, C), jnp.float32)] * 2,
    mesh=plsc.VectorSubcoreMesh(core_axis_name="c", subcore_axis_name="s"),
    compiler_params=pltpu.CompilerParams(use_tc_tiling_on_sc=False,
                                         needs_layout_passes=False),
    scratch_types=[
        pltpu.VMEM((CHUNK,), jnp.int32),
        pltpu.VMEM((CHUNK,), jnp.int32),
        pltpu.VMEM((CHUNK,), jnp.int32),
        pltpu.VMEM((CHUNK,), jnp.int32),
        pltpu.VMEM((CHUNK, C), jnp.float32),
        pltpu.VMEM((CHUNK, C), jnp.float32),
        pltpu.VMEM((N + 16,), jnp.int32),
        pltpu.VMEM_SHARED((N + 16, C), jnp.float32),
        pltpu.SemaphoreType.DMA,
        pltpu.SemaphoreType.DMA,
    ],
)(_agg_body)


_RB = 1000


def _dense_body(p0, p1, x, cnt, wl, wc, bc, ng, nb, wo, bo, o):
    cntv = cnt[...]
    deg = jnp.maximum(cntv, 1.0)
    agg = (p0[...] + p1[...]) / deg
    pre = (jnp.dot(agg, wl[...], preferred_element_type=jnp.float32)
           + jnp.dot(x[...], wc[...], preferred_element_type=jnp.float32)
           + bc[...])
    h = jnp.maximum(pre, 0.0)
    mu = jnp.mean(h, axis=1, keepdims=True)
    var = jnp.mean((h - mu) * (h - mu), axis=1, keepdims=True)
    ln = (h - mu) * lax.rsqrt(var + 1e-5) * ng[...] + nb[...]
    hsel = jnp.where(cntv > 0.0, ln, x[...])
    o[...] = jnp.dot(hsel, wo[...], preferred_element_type=jnp.float32) + bo[...]


_dense_kernel = pl.pallas_call(
    _dense_body,
    grid=(N // _RB,),
    in_specs=[
        pl.BlockSpec((_RB, C), lambda i: (i, 0)),
        pl.BlockSpec((_RB, C), lambda i: (i, 0)),
        pl.BlockSpec((_RB, C), lambda i: (i, 0)),
        pl.BlockSpec((_RB, 1), lambda i: (i, 0)),
        pl.BlockSpec((C, C), lambda i: (0, 0)),
        pl.BlockSpec((C, C), lambda i: (0, 0)),
        pl.BlockSpec((1, C), lambda i: (0, 0)),
        pl.BlockSpec((1, C), lambda i: (0, 0)),
        pl.BlockSpec((1, C), lambda i: (0, 0)),
        pl.BlockSpec((C, OUT), lambda i: (0, 0)),
        pl.BlockSpec((1, OUT), lambda i: (0, 0)),
    ],
    out_specs=pl.BlockSpec((_RB, OUT), lambda i: (i, 0)),
    out_shape=jax.ShapeDtypeStruct((N, OUT), jnp.float32),
)


def kernel(x_user, x_item, edge_user_item, edge_item_user,
           conv0_Wl, conv0_W0, conv0_W1, conv0_bl, conv0_b0, conv0_b1,
           conv0_gate, norm0_g, norm0_b,
           conv1_Wl, conv1_W0, conv1_W1, conv1_bl, conv1_b0, conv1_b1,
           conv1_gate, norm1_g, norm1_b,
           W_out, b_out):
    arp = jnp.arange(_PAD, dtype=jnp.int32)
    e0u = edge_item_user[0]
    e1u = edge_item_user[1]
    e0 = jnp.concatenate([e0u, N + (arp % 8)])
    e1 = jnp.concatenate([e1u, jnp.zeros((_PAD,), jnp.int32)])

    zc = jnp.zeros((_RT, 16), jnp.float32)
    onesc = jnp.ones((CHUNK, 16), jnp.float32)
    cs0, cs1, cd0, cd1 = _counts_kernel(e0u, e1u, zc, onesc)

    cnt_s = cs0[:, 0] + cs1[:, 0]
    cnt_d = cd0[:, 0] + cd1[:, 0]
    ps = cnt_s > 0.0
    pd = cnt_d > 0.0
    rank_s = jnp.cumsum(ps.astype(jnp.int32)) - 1
    rank_d = jnp.cumsum(pd.astype(jnp.int32)) - 1
    ar = jnp.arange(N, dtype=jnp.int32)
    nor = jnp.full((N,), N, jnp.int32).at[
        jnp.where(pd, rank_d, N)].set(ar, mode="drop")
    tmap = jnp.where(ps, nor[jnp.clip(rank_s, 0, N - 1)],
                     jnp.int32(N)).astype(jnp.int32)
    tmap = jnp.concatenate([tmap, jnp.full((16,), N, jnp.int32)])

    za = jnp.zeros((_RZ, C), jnp.float32)
    a0, a1 = _agg_kernel(e0, e1, tmap, x_user, za)

    g = jax.nn.sigmoid(conv1_gate)
    wc = (1.0 - g) * conv1_W0 + g * conv1_W1
    bc = conv1_bl + (1.0 - g) * conv1_b0 + g * conv1_b1

    return _dense_kernel(a0, a1, x_user, cnt_d[:, None],
                         conv1_Wl, wc, bc[None], norm1_g[None], norm1_b[None],
                         W_out, b_out[None])


# speedup vs baseline: 1.0562x; 1.0562x over previous
import functools

import jax
import jax.numpy as jnp
from jax import lax
from jax.experimental import pallas as pl
from jax.experimental.pallas import tpu as pltpu
from jax.experimental.pallas import tpu_sc as plsc

N = 10000
E = 320000
C = 128
OUT = 128

_NC = 2
_NS = 16
_NW = _NC * _NS
CHUNK = 128
_NCH = 80
_CH = _NW * _NCH
_EP = _CH * CHUNK
_PAD = _EP - E
_DUMP = N + 8
_RT = N // _NS
_RZ = (N + 16) // _NS
_RCH = N // CHUNK
_RREM = N - _RCH * CHUNK


def _main_body(e0_hbm, e1_hbm, x_hbm, za_hbm, zc_hbm, ones_hbm,
               a0_hbm, a1_hbm, cs0_hbm, cs1_hbm, cd0_hbm, cd1_hbm,
               q_v, e0_v, rows_v, ones_v, agg_sh, cs_sh, cd_sh, sem):
    cid = lax.axis_index("c")
    sid = lax.axis_index("s")
    wid = sid * _NC + cid
    pltpu.sync_copy(ones_hbm, ones_v)
    pltpu.sync_copy(za_hbm, agg_sh.at[pl.ds(sid * _RZ, _RZ)])
    pltpu.sync_copy(zc_hbm, cs_sh.at[pl.ds(sid * _RZ, _RZ)])
    pltpu.sync_copy(zc_hbm, cd_sh.at[pl.ds(sid * _RZ, _RZ)])
    plsc.subcore_barrier()

    def _chunk(j, carry):
        off = (j * _NW + wid) * CHUNK
        pltpu.sync_copy(e1_hbm.at[pl.ds(off, CHUNK)], q_v)
        cp = pltpu.async_copy(x_hbm.at[q_v], rows_v, sem)
        pltpu.sync_copy(e0_hbm.at[pl.ds(off, CHUNK)], e0_v)
        pltpu.sync_copy(ones_v, cs_sh.at[e0_v], add=True)
        pltpu.sync_copy(ones_v, cd_sh.at[q_v], add=True)
        cp.wait()
        pltpu.sync_copy(rows_v, agg_sh.at[e0_v], add=True)
        return carry

    lax.fori_loop(0, _NCH, _chunk, 0)
    plsc.subcore_barrier()
    ro = sid * _RT
    rz = sid * _RZ

    @pl.when(cid == 0)
    def _():
        pltpu.sync_copy(agg_sh.at[pl.ds(rz, _RZ)], a0_hbm.at[pl.ds(rz, _RZ)])
        pltpu.sync_copy(cs_sh.at[pl.ds(ro, _RT)], cs0_hbm.at[pl.ds(ro, _RT)])
        pltpu.sync_copy(cd_sh.at[pl.ds(ro, _RT)], cd0_hbm.at[pl.ds(ro, _RT)])

    @pl.when(cid == 1)
    def _():
        pltpu.sync_copy(agg_sh.at[pl.ds(rz, _RZ)], a1_hbm.at[pl.ds(rz, _RZ)])
        pltpu.sync_copy(cs_sh.at[pl.ds(ro, _RT)], cs1_hbm.at[pl.ds(ro, _RT)])
        pltpu.sync_copy(cd_sh.at[pl.ds(ro, _RT)], cd1_hbm.at[pl.ds(ro, _RT)])


_main_kernel = functools.partial(
    pl.kernel,
    out_type=[jax.ShapeDtypeStruct((N + 16, C), jnp.float32)] * 2
    + [jax.ShapeDtypeStruct((N, 16), jnp.float32)] * 4,
    mesh=plsc.VectorSubcoreMesh(core_axis_name="c", subcore_axis_name="s"),
    compiler_params=pltpu.CompilerParams(use_tc_tiling_on_sc=False,
                                         needs_layout_passes=False),
    scratch_types=[
        pltpu.VMEM((CHUNK,), jnp.int32),
        pltpu.VMEM((CHUNK,), jnp.int32),
        pltpu.VMEM((CHUNK, C), jnp.float32),
        pltpu.VMEM((CHUNK, 16), jnp.float32),
        pltpu.VMEM_SHARED((N + 16, C), jnp.float32),
        pltpu.VMEM_SHARED((N + 16, 16), jnp.float32),
        pltpu.VMEM_SHARED((N + 16, 16), jnp.float32),
        pltpu.SemaphoreType.DMA,
    ],
)(_main_body)


def _reorder_body(p0_hbm, p1_hbm, m_hbm, agg_hbm,
                  m_v, ra_v, rb_v, sa, sb):
    cid = lax.axis_index("c")
    sid = lax.axis_index("s")
    wid = sid * _NC + cid
    nch = jnp.where(wid < _RCH % _NW, _RCH // _NW + 1, _RCH // _NW)

    def _chunk(j, carry):
        off = (j * _NW + wid) * CHUNK
        pltpu.sync_copy(m_hbm.at[pl.ds(off, CHUNK)], m_v)
        cpa = pltpu.async_copy(p0_hbm.at[m_v], ra_v, sa)
        cpb = pltpu.async_copy(p1_hbm.at[m_v], rb_v, sb)
        cpa.wait()
        cpb.wait()

        def _add(i, c2):
            for k in range(C // 16):
                ra_v[i, pl.ds(k * 16, 16)] = (
                    ra_v[i, pl.ds(k * 16, 16)] + rb_v[i, pl.ds(k * 16, 16)])
            return c2

        lax.fori_loop(0, CHUNK, _add, 0)
        pltpu.sync_copy(ra_v, agg_hbm.at[pl.ds(off, CHUNK)])
        return carry

    lax.fori_loop(0, nch, _chunk, 0)

    @pl.when(wid == _RCH % _NW)
    def _():
        off = _RCH * CHUNK
        pltpu.sync_copy(m_hbm.at[pl.ds(off, _RREM)], m_v.at[pl.ds(0, _RREM)])
        cpa = pltpu.async_copy(p0_hbm.at[m_v.at[pl.ds(0, _RREM)]],
                               ra_v.at[pl.ds(0, _RREM)], sa)
        cpb = pltpu.async_copy(p1_hbm.at[m_v.at[pl.ds(0, _RREM)]],
                               rb_v.at[pl.ds(0, _RREM)], sb)
        cpa.wait()
        cpb.wait()

        def _add(i, c2):
            for k in range(C // 16):
                ra_v[i, pl.ds(k * 16, 16)] = (
                    ra_v[i, pl.ds(k * 16, 16)] + rb_v[i, pl.ds(k * 16, 16)])
            return c2

        lax.fori_loop(0, _RREM, _add, 0)
        pltpu.sync_copy(ra_v.at[pl.ds(0, _RREM)],
                        agg_hbm.at[pl.ds(off, _RREM)])


_reorder_kernel = functools.partial(
    pl.kernel,
    out_type=jax.ShapeDtypeStruct((N, C), jnp.float32),
    mesh=plsc.VectorSubcoreMesh(core_axis_name="c", subcore_axis_name="s"),
    compiler_params=pltpu.CompilerParams(use_tc_tiling_on_sc=False,
                                         needs_layout_passes=False),
    scratch_types=[
        pltpu.VMEM((CHUNK,), jnp.int32),
        pltpu.VMEM((CHUNK, C), jnp.float32),
        pltpu.VMEM((CHUNK, C), jnp.float32),
        pltpu.SemaphoreType.DMA,
        pltpu.SemaphoreType.DMA,
    ],
)(_reorder_body)


_RB = 1000


def _dense_body(p, x, cnt, wl, wc, bc, ng, nb, wo, bo, o):
    cntv = cnt[...]
    deg = jnp.maximum(cntv, 1.0)
    agg = p[...] / deg
    pre = (jnp.dot(agg, wl[...], preferred_element_type=jnp.float32)
           + jnp.dot(x[...], wc[...], preferred_element_type=jnp.float32)
           + bc[...])
    h = jnp.maximum(pre, 0.0)
    mu = jnp.mean(h, axis=1, keepdims=True)
    var = jnp.mean((h - mu) * (h - mu), axis=1, keepdims=True)
    ln = (h - mu) * lax.rsqrt(var + 1e-5) * ng[...] + nb[...]
    hsel = jnp.where(cntv > 0.0, ln, x[...])
    o[...] = jnp.dot(hsel, wo[...], preferred_element_type=jnp.float32) + bo[...]


_dense_kernel = pl.pallas_call(
    _dense_body,
    grid=(N // _RB,),
    in_specs=[
        pl.BlockSpec((_RB, C), lambda i: (i, 0)),
        pl.BlockSpec((_RB, C), lambda i: (i, 0)),
        pl.BlockSpec((_RB, 1), lambda i: (i, 0)),
        pl.BlockSpec((C, C), lambda i: (0, 0)),
        pl.BlockSpec((C, C), lambda i: (0, 0)),
        pl.BlockSpec((1, C), lambda i: (0, 0)),
        pl.BlockSpec((1, C), lambda i: (0, 0)),
        pl.BlockSpec((1, C), lambda i: (0, 0)),
        pl.BlockSpec((C, OUT), lambda i: (0, 0)),
        pl.BlockSpec((1, OUT), lambda i: (0, 0)),
    ],
    out_specs=pl.BlockSpec((_RB, OUT), lambda i: (i, 0)),
    out_shape=jax.ShapeDtypeStruct((N, OUT), jnp.float32),
)


def kernel(x_user, x_item, edge_user_item, edge_item_user,
           conv0_Wl, conv0_W0, conv0_W1, conv0_bl, conv0_b0, conv0_b1,
           conv0_gate, norm0_g, norm0_b,
           conv1_Wl, conv1_W0, conv1_W1, conv1_bl, conv1_b0, conv1_b1,
           conv1_gate, norm1_g, norm1_b,
           W_out, b_out):
    arp = jnp.arange(_PAD, dtype=jnp.int32)
    e0 = jnp.concatenate([edge_item_user[0], N + (arp % 8)])
    e1 = jnp.concatenate([edge_item_user[1], jnp.zeros((_PAD,), jnp.int32)])

    za = jnp.zeros((_RZ, C), jnp.float32)
    zc = jnp.zeros((_RZ, 16), jnp.float32)
    onesc = jnp.ones((CHUNK, 16), jnp.float32)
    p0, p1, cs0, cs1, cd0, cd1 = _main_kernel(e0, e1, x_user, za, zc, onesc)

    cnt_s = cs0[:, 0] + cs1[:, 0]
    cnt_d = cd0[:, 0] + cd1[:, 0] - jnp.where(
        jnp.arange(N) == 0, jnp.float32(_PAD), 0.0)
    ps = cnt_s > 0.0
    pd = cnt_d > 0.0
    rank_s = jnp.cumsum(ps.astype(jnp.int32)) - 1
    rank_d = jnp.cumsum(pd.astype(jnp.int32)) - 1
    ar = jnp.arange(N, dtype=jnp.int32)
    nos = jnp.full((N,), _DUMP, jnp.int32).at[
        jnp.where(ps, rank_s, N)].set(ar, mode="drop")
    m = nos[jnp.clip(rank_d, 0, N - 1)].astype(jnp.int32)

    agg = _reorder_kernel(p0, p1, m)

    g = jax.nn.sigmoid(conv1_gate)
    wc = (1.0 - g) * conv1_W0 + g * conv1_W1
    bc = conv1_bl + (1.0 - g) * conv1_b0 + g * conv1_b1

    return _dense_kernel(agg, x_user, cnt_d[:, None],
                         conv1_Wl, wc, bc[None], norm1_g[None], norm1_b[None],
                         W_out, b_out[None])


# speedup vs baseline: 1.0981x; 1.0397x over previous
import functools

import jax
import jax.numpy as jnp
from jax import lax
from jax.experimental import pallas as pl
from jax.experimental.pallas import tpu as pltpu
from jax.experimental.pallas import tpu_sc as plsc

N = 10000
E = 320000
C = 128
OUT = 128

_NC = 2
_NS = 16
_NW = _NC * _NS
CHUNK = 128
_NCH = 80
_CH = _NW * _NCH
_EP = _CH * CHUNK
_PAD = _EP - E
_DUMP = N + 8
_RT = N // _NS
_RZ = (N + 16) // _NS
_RCH = N // CHUNK
_RREM = N - _RCH * CHUNK


_NP = N + 16


def _main_body(e0_hbm, e1_hbm, x_hbm, za_hbm, zc_hbm,
               a0_hbm, a1_hbm, csl_hbm, cdl_hbm,
               q_v, e0_v, rows_v, cs_loc, cd_loc, agg_sh, sem):
    cid = lax.axis_index("c")
    sid = lax.axis_index("s")
    wid = sid * _NC + cid
    one16 = jnp.ones((16,), jnp.float32)
    pltpu.sync_copy(zc_hbm, cs_loc)
    pltpu.sync_copy(zc_hbm, cd_loc)
    pltpu.sync_copy(za_hbm, agg_sh.at[pl.ds(sid * _RZ, _RZ)])
    plsc.subcore_barrier()

    def _chunk(j, carry):
        off = (j * _NW + wid) * CHUNK
        pltpu.sync_copy(e1_hbm.at[pl.ds(off, CHUNK)], q_v)
        cp = pltpu.async_copy(x_hbm.at[q_v], rows_v, sem)
        pltpu.sync_copy(e0_hbm.at[pl.ds(off, CHUNK)], e0_v)

        def _cnt(i, c2):
            plsc.addupdate_scatter(cs_loc, [e0_v[pl.ds(i * 16, 16)]], one16)
            plsc.addupdate_scatter(cd_loc, [q_v[pl.ds(i * 16, 16)]], one16)
            return c2

        lax.fori_loop(0, CHUNK // 16, _cnt, 0)
        cp.wait()
        pltpu.sync_copy(rows_v, agg_sh.at[e0_v], add=True)
        return carry

    lax.fori_loop(0, _NCH, _chunk, 0)
    plsc.subcore_barrier()
    rz = sid * _RZ
    pltpu.sync_copy(cs_loc, csl_hbm.at[pl.ds(wid * _NP, _NP)])
    pltpu.sync_copy(cd_loc, cdl_hbm.at[pl.ds(wid * _NP, _NP)])

    @pl.when(cid == 0)
    def _():
        pltpu.sync_copy(agg_sh.at[pl.ds(rz, _RZ)], a0_hbm.at[pl.ds(rz, _RZ)])

    @pl.when(cid == 1)
    def _():
        pltpu.sync_copy(agg_sh.at[pl.ds(rz, _RZ)], a1_hbm.at[pl.ds(rz, _RZ)])


_main_kernel = functools.partial(
    pl.kernel,
    out_type=[jax.ShapeDtypeStruct((N + 16, C), jnp.float32)] * 2
    + [jax.ShapeDtypeStruct((_NW * _NP,), jnp.float32)] * 2,
    mesh=plsc.VectorSubcoreMesh(core_axis_name="c", subcore_axis_name="s"),
    compiler_params=pltpu.CompilerParams(use_tc_tiling_on_sc=False,
                                         needs_layout_passes=False),
    scratch_types=[
        pltpu.VMEM((CHUNK,), jnp.int32),
        pltpu.VMEM((CHUNK,), jnp.int32),
        pltpu.VMEM((CHUNK, C), jnp.float32),
        pltpu.VMEM((_NP,), jnp.float32),
        pltpu.VMEM((_NP,), jnp.float32),
        pltpu.VMEM_SHARED((N + 16, C), jnp.float32),
        pltpu.SemaphoreType.DMA,
    ],
)(_main_body)


def _reorder_body(p0_hbm, p1_hbm, m_hbm, agg_hbm,
                  m_v, ra_v, rb_v, sa, sb):
    cid = lax.axis_index("c")
    sid = lax.axis_index("s")
    wid = sid * _NC + cid
    nch = jnp.where(wid < _RCH % _NW, _RCH // _NW + 1, _RCH // _NW)

    def _chunk(j, carry):
        off = (j * _NW + wid) * CHUNK
        pltpu.sync_copy(m_hbm.at[pl.ds(off, CHUNK)], m_v)
        cpa = pltpu.async_copy(p0_hbm.at[m_v], ra_v, sa)
        cpb = pltpu.async_copy(p1_hbm.at[m_v], rb_v, sb)
        cpa.wait()
        cpb.wait()

        def _add(i, c2):
            for k in range(C // 16):
                ra_v[i, pl.ds(k * 16, 16)] = (
                    ra_v[i, pl.ds(k * 16, 16)] + rb_v[i, pl.ds(k * 16, 16)])
            return c2

        lax.fori_loop(0, CHUNK, _add, 0)
        pltpu.sync_copy(ra_v, agg_hbm.at[pl.ds(off, CHUNK)])
        return carry

    lax.fori_loop(0, nch, _chunk, 0)

    @pl.when(wid == _RCH % _NW)
    def _():
        off = _RCH * CHUNK
        pltpu.sync_copy(m_hbm.at[pl.ds(off, _RREM)], m_v.at[pl.ds(0, _RREM)])
        cpa = pltpu.async_copy(p0_hbm.at[m_v.at[pl.ds(0, _RREM)]],
                               ra_v.at[pl.ds(0, _RREM)], sa)
        cpb = pltpu.async_copy(p1_hbm.at[m_v.at[pl.ds(0, _RREM)]],
                               rb_v.at[pl.ds(0, _RREM)], sb)
        cpa.wait()
        cpb.wait()

        def _add(i, c2):
            for k in range(C // 16):
                ra_v[i, pl.ds(k * 16, 16)] = (
                    ra_v[i, pl.ds(k * 16, 16)] + rb_v[i, pl.ds(k * 16, 16)])
            return c2

        lax.fori_loop(0, _RREM, _add, 0)
        pltpu.sync_copy(ra_v.at[pl.ds(0, _RREM)],
                        agg_hbm.at[pl.ds(off, _RREM)])


_reorder_kernel = functools.partial(
    pl.kernel,
    out_type=jax.ShapeDtypeStruct((N, C), jnp.float32),
    mesh=plsc.VectorSubcoreMesh(core_axis_name="c", subcore_axis_name="s"),
    compiler_params=pltpu.CompilerParams(use_tc_tiling_on_sc=False,
                                         needs_layout_passes=False),
    scratch_types=[
        pltpu.VMEM((CHUNK,), jnp.int32),
        pltpu.VMEM((CHUNK, C), jnp.float32),
        pltpu.VMEM((CHUNK, C), jnp.float32),
        pltpu.SemaphoreType.DMA,
        pltpu.SemaphoreType.DMA,
    ],
)(_reorder_body)


_RB = 1000


def _dense_body(p, x, cnt, wl, wc, bc, ng, nb, wo, bo, o):
    cntv = cnt[...]
    deg = jnp.maximum(cntv, 1.0)
    agg = p[...] / deg
    pre = (jnp.dot(agg, wl[...], preferred_element_type=jnp.float32)
           + jnp.dot(x[...], wc[...], preferred_element_type=jnp.float32)
           + bc[...])
    h = jnp.maximum(pre, 0.0)
    mu = jnp.mean(h, axis=1, keepdims=True)
    var = jnp.mean((h - mu) * (h - mu), axis=1, keepdims=True)
    ln = (h - mu) * lax.rsqrt(var + 1e-5) * ng[...] + nb[...]
    hsel = jnp.where(cntv > 0.0, ln, x[...])
    o[...] = jnp.dot(hsel, wo[...], preferred_element_type=jnp.float32) + bo[...]


_dense_kernel = pl.pallas_call(
    _dense_body,
    grid=(N // _RB,),
    in_specs=[
        pl.BlockSpec((_RB, C), lambda i: (i, 0)),
        pl.BlockSpec((_RB, C), lambda i: (i, 0)),
        pl.BlockSpec((_RB, 1), lambda i: (i, 0)),
        pl.BlockSpec((C, C), lambda i: (0, 0)),
        pl.BlockSpec((C, C), lambda i: (0, 0)),
        pl.BlockSpec((1, C), lambda i: (0, 0)),
        pl.BlockSpec((1, C), lambda i: (0, 0)),
        pl.BlockSpec((1, C), lambda i: (0, 0)),
        pl.BlockSpec((C, OUT), lambda i: (0, 0)),
        pl.BlockSpec((1, OUT), lambda i: (0, 0)),
    ],
    out_specs=pl.BlockSpec((_RB, OUT), lambda i: (i, 0)),
    out_shape=jax.ShapeDtypeStruct((N, OUT), jnp.float32),
)


def kernel(x_user, x_item, edge_user_item, edge_item_user,
           conv0_Wl, conv0_W0, conv0_W1, conv0_bl, conv0_b0, conv0_b1,
           conv0_gate, norm0_g, norm0_b,
           conv1_Wl, conv1_W0, conv1_W1, conv1_bl, conv1_b0, conv1_b1,
           conv1_gate, norm1_g, norm1_b,
           W_out, b_out):
    arp = jnp.arange(_PAD, dtype=jnp.int32)
    e0 = jnp.concatenate([edge_item_user[0], N + (arp % 8)])
    e1 = jnp.concatenate([edge_item_user[1], jnp.zeros((_PAD,), jnp.int32)])

    za = jnp.zeros((_RZ, C), jnp.float32)
    zc = jnp.zeros((_NP,), jnp.float32)
    p0, p1, csl, cdl = _main_kernel(e0, e1, x_user, za, zc)

    cnt_s = csl.reshape(_NW, _NP).sum(0)[:N]
    cnt_d = cdl.reshape(_NW, _NP).sum(0)[:N] - jnp.where(
        jnp.arange(N) == 0, jnp.float32(_PAD), 0.0)
    ps = cnt_s > 0.0
    pd = cnt_d > 0.0
    rank_s = jnp.cumsum(ps.astype(jnp.int32)) - 1
    rank_d = jnp.cumsum(pd.astype(jnp.int32)) - 1
    ar = jnp.arange(N, dtype=jnp.int32)
    nos = jnp.full((N,), _DUMP, jnp.int32).at[
        jnp.where(ps, rank_s, N)].set(ar, mode="drop")
    m = nos[jnp.clip(rank_d, 0, N - 1)].astype(jnp.int32)

    agg = _reorder_kernel(p0, p1, m)

    g = jax.nn.sigmoid(conv1_gate)
    wc = (1.0 - g) * conv1_W0 + g * conv1_W1
    bc = conv1_bl + (1.0 - g) * conv1_b0 + g * conv1_b1

    return _dense_kernel(agg, x_user, cnt_d[:, None],
                         conv1_Wl, wc, bc[None], norm1_g[None], norm1_b[None],
                         W_out, b_out[None])


# speedup vs baseline: 1.6474x; 1.5002x over previous
import functools

import jax
import jax.numpy as jnp
from jax import lax
from jax.experimental import pallas as pl
from jax.experimental.pallas import tpu as pltpu
from jax.experimental.pallas import tpu_sc as plsc

N = 10000
E = 320000
C = 128
OUT = 128

_NC = 2
_NS = 16
_NW = _NC * _NS
CHUNK = 128
_CH = E // CHUNK
_CHB = _CH // _NW
_CHR = _CH % _NW
_RT = N // _NS
_RZ = (N + 16) // _NS


_NP = N + 16


def _counts_body(e0_hbm, e1_hbm, zc_hbm,
                 csl_hbm, cdl_hbm,
                 e0_v, e1_v, cs_loc, cd_loc):
    cid = lax.axis_index("c")
    sid = lax.axis_index("s")
    wid = sid * _NC + cid
    one16 = jnp.ones((16,), jnp.float32)
    pltpu.sync_copy(zc_hbm, cs_loc)
    pltpu.sync_copy(zc_hbm, cd_loc)
    nch = jnp.where(wid < _CHR, _CHB + 1, _CHB)

    def _chunk(j, carry):
        off = (wid + j * _NW) * CHUNK
        pltpu.sync_copy(e0_hbm.at[pl.ds(off, CHUNK)], e0_v)
        pltpu.sync_copy(e1_hbm.at[pl.ds(off, CHUNK)], e1_v)

        def _cnt(i, c2):
            plsc.addupdate_scatter(cs_loc, [e0_v[pl.ds(i * 16, 16)]], one16)
            plsc.addupdate_scatter(cd_loc, [e1_v[pl.ds(i * 16, 16)]], one16)
            return c2

        lax.fori_loop(0, CHUNK // 16, _cnt, 0)
        return carry

    lax.fori_loop(0, nch, _chunk, 0)
    pltpu.sync_copy(cs_loc, csl_hbm.at[pl.ds(wid * _NP, _NP)])
    pltpu.sync_copy(cd_loc, cdl_hbm.at[pl.ds(wid * _NP, _NP)])


_counts_kernel = functools.partial(
    pl.kernel,
    out_type=[jax.ShapeDtypeStruct((_NW * _NP,), jnp.float32)] * 2,
    mesh=plsc.VectorSubcoreMesh(core_axis_name="c", subcore_axis_name="s"),
    compiler_params=pltpu.CompilerParams(use_tc_tiling_on_sc=False, needs_layout_passes=False),
    scratch_types=[
        pltpu.VMEM((CHUNK,), jnp.int32),
        pltpu.VMEM((CHUNK,), jnp.int32),
        pltpu.VMEM((_NP,), jnp.float32),
        pltpu.VMEM((_NP,), jnp.float32),
    ],
)(_counts_body)


def _agg_body(e0_hbm, e1_hbm, t_hbm, x_hbm, z_hbm,
              a0_hbm, a1_hbm,
              e0_v, e1_v, tgt_v, rows_v, t_v, agg_sh, sem):
    cid = lax.axis_index("c")
    sid = lax.axis_index("s")
    wid = sid * _NC + cid
    pltpu.sync_copy(t_hbm, t_v)
    pltpu.sync_copy(z_hbm, agg_sh.at[pl.ds(sid * _RZ, _RZ)])
    plsc.subcore_barrier()
    nch = jnp.where(wid < _CHR, _CHB + 1, _CHB)

    def _chunk(j, carry):
        off = (wid + j * _NW) * CHUNK
        pltpu.sync_copy(e1_hbm.at[pl.ds(off, CHUNK)], e1_v)
        cp = pltpu.async_copy(x_hbm.at[e1_v], rows_v, sem)
        pltpu.sync_copy(e0_hbm.at[pl.ds(off, CHUNK)], e0_v)

        def _map(i, c2):
            idx16 = e0_v[pl.ds(i * 16, 16)]
            tgt_v[pl.ds(i * 16, 16)] = plsc.load_gather(t_v, [idx16])
            return c2

        lax.fori_loop(0, CHUNK // 16, _map, 0)
        cp.wait()
        pltpu.sync_copy(rows_v, agg_sh.at[tgt_v], add=True)
        return carry

    lax.fori_loop(0, nch, _chunk, 0)
    plsc.subcore_barrier()
    r0 = sid * _RT

    @pl.when(cid == 0)
    def _():
        pltpu.sync_copy(agg_sh.at[pl.ds(r0, _RT)], a0_hbm.at[pl.ds(r0, _RT)])

    @pl.when(cid == 1)
    def _():
        pltpu.sync_copy(agg_sh.at[pl.ds(r0, _RT)], a1_hbm.at[pl.ds(r0, _RT)])


_agg_kernel = functools.partial(
    pl.kernel,
    out_type=[jax.ShapeDtypeStruct((N, C), jnp.float32)] * 2,
    mesh=plsc.VectorSubcoreMesh(core_axis_name="c", subcore_axis_name="s"),
    compiler_params=pltpu.CompilerParams(use_tc_tiling_on_sc=False, needs_layout_passes=False),
    scratch_types=[
        pltpu.VMEM((CHUNK,), jnp.int32),
        pltpu.VMEM((CHUNK,), jnp.int32),
        pltpu.VMEM((CHUNK,), jnp.int32),
        pltpu.VMEM((CHUNK, C), jnp.float32),
        pltpu.VMEM((N,), jnp.int32),
        pltpu.VMEM_SHARED((N + 16, C), jnp.float32),
        pltpu.SemaphoreType.DMA,
    ],
)(_agg_body)


_RB = 1000


def _dense_body(p0, p1, x, cnt, wl, wc, bc, ng, nb, wo, bo, o):
    cntv = cnt[...]
    deg = jnp.maximum(cntv, 1.0)
    agg = (p0[...] + p1[...]) / deg
    pre = (jnp.dot(agg, wl[...], preferred_element_type=jnp.float32)
           + jnp.dot(x[...], wc[...], preferred_element_type=jnp.float32)
           + bc[...])
    h = jnp.maximum(pre, 0.0)
    mu = jnp.mean(h, axis=1, keepdims=True)
    var = jnp.mean((h - mu) * (h - mu), axis=1, keepdims=True)
    ln = (h - mu) * lax.rsqrt(var + 1e-5) * ng[...] + nb[...]
    hsel = jnp.where(cntv > 0.0, ln, x[...])
    o[...] = jnp.dot(hsel, wo[...], preferred_element_type=jnp.float32) + bo[...]


_dense_kernel = pl.pallas_call(
    _dense_body,
    grid=(N // _RB,),
    in_specs=[
        pl.BlockSpec((_RB, C), lambda i: (i, 0)),
        pl.BlockSpec((_RB, C), lambda i: (i, 0)),
        pl.BlockSpec((_RB, C), lambda i: (i, 0)),
        pl.BlockSpec((_RB, 1), lambda i: (i, 0)),
        pl.BlockSpec((C, C), lambda i: (0, 0)),
        pl.BlockSpec((C, C), lambda i: (0, 0)),
        pl.BlockSpec((1, C), lambda i: (0, 0)),
        pl.BlockSpec((1, C), lambda i: (0, 0)),
        pl.BlockSpec((1, C), lambda i: (0, 0)),
        pl.BlockSpec((C, OUT), lambda i: (0, 0)),
        pl.BlockSpec((1, OUT), lambda i: (0, 0)),
    ],
    out_specs=pl.BlockSpec((_RB, OUT), lambda i: (i, 0)),
    out_shape=jax.ShapeDtypeStruct((N, OUT), jnp.float32),
)


def kernel(x_user, x_item, edge_user_item, edge_item_user,
           conv0_Wl, conv0_W0, conv0_W1, conv0_bl, conv0_b0, conv0_b1,
           conv0_gate, norm0_g, norm0_b,
           conv1_Wl, conv1_W0, conv1_W1, conv1_bl, conv1_b0, conv1_b1,
           conv1_gate, norm1_g, norm1_b,
           W_out, b_out):
    e0 = edge_item_user[0]
    e1 = edge_item_user[1]

    zc = jnp.zeros((_NP,), jnp.float32)
    csl, cdl = _counts_kernel(e0, e1, zc)

    cnt_s = csl.reshape(_NW, _NP).sum(0)[:N]
    cnt_d = cdl.reshape(_NW, _NP).sum(0)[:N]
    ps = cnt_s > 0.0
    pd = cnt_d > 0.0
    rank_s = jnp.cumsum(ps.astype(jnp.int32)) - 1
    rank_d = jnp.cumsum(pd.astype(jnp.int32)) - 1
    ar = jnp.arange(N, dtype=jnp.int32)
    nor = jnp.full((N,), N, jnp.int32).at[
        jnp.where(pd, rank_d, N)].set(ar, mode="drop")
    tmap = jnp.where(ps, nor[jnp.clip(rank_s, 0, N - 1)],
                     jnp.int32(N)).astype(jnp.int32)

    za = jnp.zeros((_RZ, C), jnp.float32)
    a0, a1 = _agg_kernel(e0, e1, tmap, x_user, za)

    g = jax.nn.sigmoid(conv1_gate)
    wc = (1.0 - g) * conv1_W0 + g * conv1_W1
    bc = conv1_bl + (1.0 - g) * conv1_b0 + g * conv1_b1

    return _dense_kernel(a0, a1, x_user, cnt_d[:, None],
                         conv1_Wl, wc, bc[None], norm1_g[None], norm1_b[None],
                         W_out, b_out[None])
